# Initial kernel scaffold; baseline (speedup 1.0000x reference)
import jax
import jax.numpy as jnp
from jax.experimental import pallas as pl


def kernel(edge_index, edge_type, emb, comp1, bases1, root1, bias1, comp2, bases2, root2, bias2):
    raise NotImplementedError("write your pallas kernel here")


# speedup vs baseline: 37.7944x; 37.7944x over previous
import functools

import jax
import jax.numpy as jnp
from jax import lax
from jax.experimental import pallas as pl
from jax.experimental.pallas import tpu as pltpu
from jax.experimental.pallas import tpu_sc as plsc

N = 10000
R = 8
D = 128
E = 320000
NB = 8
NC = 2
NS = 16
NW = NC * NS
L = 16

CE = 80
NCH = (E // NW) // CE
NP = 10240
ROWS_PER_SUB = NP // NS
CNT = R * N
CNTP = 81920
CNT_SLAB = CNTP // NS


def _mm_body(x_ref, compf_ref, basesf_ref, bias_ref, out_ref):
    r = pl.program_id(1)
    crow = compf_ref[0, 0, :]
    w = jnp.sum(crow[:, None, None] * basesf_ref[...], axis=0)
    y = jnp.dot(x_ref[...], w, preferred_element_type=jnp.float32)
    flag = jnp.where(r == R, 1.0, 0.0)
    out_ref[0] = y + flag * bias_ref[...]


def _matmul_stage(x, compf3, basesf, bias2):
    nblk = 10
    blk = N // nblk
    return pl.pallas_call(
        _mm_body,
        grid=(nblk, R + 1),
        in_specs=[
            pl.BlockSpec((blk, D), lambda i, r: (i, 0)),
            pl.BlockSpec((1, 1, R + 1), lambda i, r: (r, 0, 0)),
            pl.BlockSpec((R + 1, D, D), lambda i, r: (0, 0, 0)),
            pl.BlockSpec((1, D), lambda i, r: (0, 0)),
        ],
        out_specs=pl.BlockSpec((1, blk, D), lambda i, r: (r, i, 0)),
        out_shape=jax.ShapeDtypeStruct((R + 1, N, D), jnp.float32),
    )(x, compf3, basesf, bias2)


def _relu_body(base_ref, parts_ref, out_ref):
    out_ref[...] = jnp.maximum(base_ref[0] + parts_ref[0] + parts_ref[1], 0.0)


def _relu_stage(hfull, parts):
    nblk = 10
    blk = N // nblk
    return pl.pallas_call(
        _relu_body,
        grid=(nblk,),
        in_specs=[
            pl.BlockSpec((1, blk, D), lambda i: (R, i, 0)),
            pl.BlockSpec((NC, blk, D), lambda i: (0, i, 0)),
        ],
        out_specs=pl.BlockSpec((blk, D), lambda i: (i, 0)),
        out_shape=jax.ShapeDtypeStruct((N, D), jnp.float32),
    )(hfull, parts)


def _weights_body(fdst_hbm, zeros_hbm, w_hbm,
                  rid_v, wbuf_v, ones_v, cgt_v, zb_v, cnt_sh):
    cid = lax.axis_index("c")
    sid = lax.axis_index("s")
    wid = sid * NC + cid

    pltpu.sync_copy(zeros_hbm.at[pl.ds(sid * CNT_SLAB, CNT_SLAB)], zb_v)
    pltpu.sync_copy(zb_v, cnt_sh.at[pl.ds(sid * CNT_SLAB, CNT_SLAB)])
    for g in range(CE // L):
        ones_v[pl.ds(g * L, L)] = jnp.ones((L,), jnp.float32)
    plsc.subcore_barrier()

    def count_chunk(ch, _):
        pltpu.sync_copy(ones_v, cnt_sh.at[rid_v.at[ch]], add=True)
        return _

    for k in range(2):
        pltpu.sync_copy(fdst_hbm.at[2 * sid + k], rid_v)
        lax.fori_loop(0, NCH, count_chunk, None)
    plsc.subcore_barrier()

    pltpu.sync_copy(fdst_hbm.at[wid], rid_v)

    def weight_chunk(ch, _):
        pltpu.sync_copy(cnt_sh.at[rid_v.at[ch]], cgt_v)
        for g in range(CE // L):
            cnt16 = cgt_v[pl.ds(g * L, L)]
            wbuf_v[ch, pl.ds(g * L, L)] = 1.0 / jnp.maximum(cnt16, 1.0)
        return _

    lax.fori_loop(0, NCH, weight_chunk, None)
    pltpu.sync_copy(wbuf_v, w_hbm.at[wid])


def _edge_weights(fdst3, zeros_cnt):
    mesh = plsc.VectorSubcoreMesh(core_axis_name="c", subcore_axis_name="s")
    return pl.kernel(
        _weights_body,
        out_type=jax.ShapeDtypeStruct((NW, NCH, CE), jnp.float32),
        mesh=mesh,
        compiler_params=pltpu.CompilerParams(use_tc_tiling_on_sc=False),
        scratch_types=[
            pltpu.VMEM((NCH, CE), jnp.int32),
            pltpu.VMEM((NCH, CE), jnp.float32),
            pltpu.VMEM((CE,), jnp.float32),
            pltpu.VMEM((CE,), jnp.float32),
            pltpu.VMEM((CNT_SLAB,), jnp.float32),
            pltpu.VMEM_SHARED((CNTP,), jnp.float32),
        ],
    )(fdst3, zeros_cnt)


_BCAST_DNUMS = lax.GatherDimensionNumbers(
    offset_dims=(), collapsed_slice_dims=(0,), start_index_map=(0,))


def _bcast_lane(v, j):
    idx = jnp.full((L, 1), j, jnp.int32)
    return lax.gather(v, idx, _BCAST_DNUMS, (1,),
                      mode=lax.GatherScatterMode.PROMISE_IN_BOUNDS)


def _scatter_body(h_hbm, fsrc_hbm, dst_hbm, w_hbm, zeros_hbm, parts_hbm,
                  fsrc_v, dst_v, w_v, rows_v, acc_sh, sem):
    cid = lax.axis_index("c")
    sid = lax.axis_index("s")
    wid = sid * NC + cid
    row0 = sid * ROWS_PER_SUB

    pltpu.sync_copy(zeros_hbm, rows_v)
    for k in range(ROWS_PER_SUB // CE):
        pltpu.sync_copy(rows_v, acc_sh.at[pl.ds(row0 + k * CE, CE), :])

    pltpu.sync_copy(fsrc_hbm.at[wid], fsrc_v)
    pltpu.sync_copy(dst_hbm.at[wid], dst_v)
    pltpu.sync_copy(w_hbm.at[wid], w_v)
    plsc.subcore_barrier()

    def chunk(ch, _):
        pltpu.async_copy(h_hbm.at[fsrc_v.at[ch]], rows_v, sem).wait()

        def scale_block(b, _c):
            w16 = w_v[ch, pl.ds(b * L, L)]
            for j in range(L):
                e = b * L + j
                wbc = _bcast_lane(w16, j)
                for t in range(D // L):
                    rows_v[e, pl.ds(t * L, L)] = rows_v[e, pl.ds(t * L, L)] * wbc
            return _c

        lax.fori_loop(0, CE // L, scale_block, None)
        pltpu.sync_copy(rows_v, acc_sh.at[dst_v.at[ch]], add=True)
        return _

    lax.fori_loop(0, NCH, chunk, None)
    plsc.subcore_barrier()

    for k in range(ROWS_PER_SUB // CE):
        pltpu.sync_copy(acc_sh.at[pl.ds(row0 + k * CE, CE), :], rows_v)
        pltpu.sync_copy(rows_v, parts_hbm.at[cid, pl.ds(row0 + k * CE, CE), :])


def _edge_scatter(h2d, fsrc3, dst3, w3, zeros_rows):
    mesh = plsc.VectorSubcoreMesh(core_axis_name="c", subcore_axis_name="s")
    return pl.kernel(
        _scatter_body,
        out_type=jax.ShapeDtypeStruct((NC, NP, D), jnp.float32),
        mesh=mesh,
        compiler_params=pltpu.CompilerParams(use_tc_tiling_on_sc=False),
        scratch_types=[
            pltpu.VMEM((NCH, CE), jnp.int32),
            pltpu.VMEM((NCH, CE), jnp.int32),
            pltpu.VMEM((NCH, CE), jnp.float32),
            pltpu.VMEM((CE, D), jnp.float32),
            pltpu.VMEM_SHARED((NP, D), jnp.float32),
            pltpu.SemaphoreType.DMA,
        ],
    )(h2d, fsrc3, dst3, w3, zeros_rows)


@jax.jit
def kernel(edge_index, edge_type, emb, comp1, bases1, root1, bias1,
           comp2, bases2, root2, bias2):
    src = edge_index[0]
    dst = edge_index[1]

    fsrc3 = (edge_type * N + src).reshape(NW, NCH, CE)
    fdst3 = (edge_type * N + dst).reshape(NW, NCH, CE)
    dst3 = dst.reshape(NW, NCH, CE)

    zeros_cnt = jnp.zeros((CNTP,), jnp.float32)
    zeros_rows = jnp.zeros((CE, D), jnp.float32)

    w3 = _edge_weights(fdst3, zeros_cnt)

    def layer(x, comp, bases, root, bias):
        compf = jnp.zeros((R + 1, R + 1), jnp.float32)
        compf = compf.at[:R, :NB].set(comp).at[R, R].set(1.0)
        basesf = jnp.concatenate([bases, root[None]], axis=0)
        hfull = _matmul_stage(x, compf.reshape(R + 1, 1, R + 1), basesf,
                              bias.reshape(1, D))
        h2d = hfull.reshape((R + 1) * N, D)
        parts = _edge_scatter(h2d, fsrc3, dst3, w3, zeros_rows)
        return _relu_stage(hfull, parts)

    x = layer(emb, comp1, bases1, root1, bias1)
    x = layer(x, comp2, bases2, root2, bias2)
    return x


# speedup vs baseline: 43.7021x; 1.1563x over previous
import functools

import jax
import jax.numpy as jnp
from jax import lax
from jax.experimental import pallas as pl
from jax.experimental.pallas import tpu as pltpu
from jax.experimental.pallas import tpu_sc as plsc

N = 10000
R = 8
D = 128
E = 320000
NB = 8
NC = 2
NS = 16
NW = NC * NS
L = 16

CE = 80
NCH = (E // NW) // CE
NP = 10000
ROWS_PER_SUB = NP // NS
CNT = R * N
CNTP = 81920
CNT_SLAB = CNTP // NS


def _mm_body(x_ref, compf_ref, basesf_ref, bias_ref, out_ref):
    r = pl.program_id(1)
    crow = compf_ref[0, 0, :]
    w = jnp.sum(crow[:, None, None] * basesf_ref[...], axis=0)
    y = jnp.dot(x_ref[...], w, preferred_element_type=jnp.float32)
    flag = jnp.where(r == R, 1.0, 0.0)
    out_ref[0] = y + flag * bias_ref[...]


def _matmul_stage(x, compf3, basesf, bias2):
    nblk = 10
    blk = N // nblk
    return pl.pallas_call(
        _mm_body,
        grid=(nblk, R + 1),
        in_specs=[
            pl.BlockSpec((blk, D), lambda i, r: (i, 0)),
            pl.BlockSpec((1, 1, R + 1), lambda i, r: (r, 0, 0)),
            pl.BlockSpec((R + 1, D, D), lambda i, r: (0, 0, 0)),
            pl.BlockSpec((1, D), lambda i, r: (0, 0)),
        ],
        out_specs=pl.BlockSpec((1, blk, D), lambda i, r: (r, i, 0)),
        out_shape=jax.ShapeDtypeStruct((R + 1, N, D), jnp.float32),
    )(x, compf3, basesf, bias2)


def _relu_body(base_ref, parts_ref, out_ref):
    out_ref[...] = jnp.maximum(base_ref[0] + parts_ref[0] + parts_ref[1], 0.0)


def _relu_stage(hfull, parts):
    nblk = 10
    blk = N // nblk
    return pl.pallas_call(
        _relu_body,
        grid=(nblk,),
        in_specs=[
            pl.BlockSpec((1, blk, D), lambda i: (R, i, 0)),
            pl.BlockSpec((NC, blk, D), lambda i: (0, i, 0)),
        ],
        out_specs=pl.BlockSpec((blk, D), lambda i: (i, 0)),
        out_shape=jax.ShapeDtypeStruct((N, D), jnp.float32),
    )(hfull, parts)


def _weights_body(fdst_hbm, zeros_hbm, w_hbm,
                  rid_v, wbuf_v, ones_v, cgt_v, zb_v, cnt_sh):
    cid = lax.axis_index("c")
    sid = lax.axis_index("s")
    wid = sid * NC + cid

    pltpu.sync_copy(zeros_hbm.at[pl.ds(sid * CNT_SLAB, CNT_SLAB)], zb_v)
    pltpu.sync_copy(zb_v, cnt_sh.at[pl.ds(sid * CNT_SLAB, CNT_SLAB)])
    for g in range(CE // L):
        ones_v[pl.ds(g * L, L)] = jnp.ones((L,), jnp.float32)
    plsc.subcore_barrier()

    def count_chunk(ch, _):
        pltpu.sync_copy(ones_v, cnt_sh.at[rid_v.at[ch]], add=True)
        return _

    for k in range(2):
        pltpu.sync_copy(fdst_hbm.at[2 * sid + k], rid_v)
        lax.fori_loop(0, NCH, count_chunk, None)
    plsc.subcore_barrier()

    pltpu.sync_copy(fdst_hbm.at[wid], rid_v)

    def weight_chunk(ch, _):
        pltpu.sync_copy(cnt_sh.at[rid_v.at[ch]], cgt_v)
        for g in range(CE // L):
            cnt16 = cgt_v[pl.ds(g * L, L)]
            wbuf_v[ch, pl.ds(g * L, L)] = 1.0 / jnp.maximum(cnt16, 1.0)
        return _

    lax.fori_loop(0, NCH, weight_chunk, None)
    pltpu.sync_copy(wbuf_v, w_hbm.at[wid])


def _edge_weights(fdst3, zeros_cnt):
    mesh = plsc.VectorSubcoreMesh(core_axis_name="c", subcore_axis_name="s")
    return pl.kernel(
        _weights_body,
        out_type=jax.ShapeDtypeStruct((NW, NCH, CE), jnp.float32),
        mesh=mesh,
        compiler_params=pltpu.CompilerParams(use_tc_tiling_on_sc=False),
        scratch_types=[
            pltpu.VMEM((NCH, CE), jnp.int32),
            pltpu.VMEM((NCH, CE), jnp.float32),
            pltpu.VMEM((CE,), jnp.float32),
            pltpu.VMEM((CE,), jnp.float32),
            pltpu.VMEM((CNT_SLAB,), jnp.float32),
            pltpu.VMEM_SHARED((CNTP,), jnp.float32),
        ],
    )(fdst3, zeros_cnt)


_BCAST_DNUMS = lax.GatherDimensionNumbers(
    offset_dims=(), collapsed_slice_dims=(0,), start_index_map=(0,))


def _bcast_lane(v, j):
    idx = jnp.full((L, 1), j, jnp.int32)
    return lax.gather(v, idx, _BCAST_DNUMS, (1,),
                      mode=lax.GatherScatterMode.PROMISE_IN_BOUNDS)


def _scatter_body(h_hbm, fsrc_hbm, dst_hbm, w_hbm, zeros_hbm, parts_hbm,
                  fsrc_v, dst_v, w_v, rows_v, rows2_v, acc_sh, sem, sem2):
    cid = lax.axis_index("c")
    sid = lax.axis_index("s")
    wid = sid * NC + cid
    row0 = sid * ROWS_PER_SUB

    pltpu.sync_copy(zeros_hbm, rows_v)
    for k in range(ROWS_PER_SUB // CE):
        pltpu.sync_copy(rows_v, acc_sh.at[pl.ds(row0 + k * CE, CE), :])
    tail0 = row0 + (ROWS_PER_SUB // CE) * CE
    tail_n = ROWS_PER_SUB - (ROWS_PER_SUB // CE) * CE
    pltpu.sync_copy(rows_v.at[pl.ds(0, tail_n)],
                    acc_sh.at[pl.ds(tail0, tail_n), :])

    pltpu.sync_copy(fsrc_hbm.at[wid], fsrc_v)
    pltpu.sync_copy(dst_hbm.at[wid], dst_v)
    pltpu.sync_copy(w_hbm.at[wid], w_v)
    plsc.subcore_barrier()

    def scale_scatter(ch, buf):
        def scale_block(b, _c):
            w16 = w_v[ch, pl.ds(b * L, L)]
            for j in range(L):
                e = b * L + j
                wbc = _bcast_lane(w16, j)
                for t in range(D // L):
                    buf[e, pl.ds(t * L, L)] = buf[e, pl.ds(t * L, L)] * wbc
            return _c

        lax.fori_loop(0, CE // L, scale_block, None)
        pltpu.sync_copy(buf, acc_sh.at[dst_v.at[ch]], add=True)

    def pair(i, _):
        ch0 = 2 * i
        c0 = pltpu.async_copy(h_hbm.at[fsrc_v.at[ch0]], rows_v, sem)
        c1 = pltpu.async_copy(h_hbm.at[fsrc_v.at[ch0 + 1]], rows2_v, sem2)
        c0.wait()
        scale_scatter(ch0, rows_v)
        c1.wait()
        scale_scatter(ch0 + 1, rows2_v)
        return _

    lax.fori_loop(0, NCH // 2, pair, None)
    pltpu.async_copy(h_hbm.at[fsrc_v.at[NCH - 1]], rows_v, sem).wait()
    scale_scatter(NCH - 1, rows_v)
    plsc.subcore_barrier()

    for k in range(ROWS_PER_SUB // CE):
        pltpu.sync_copy(acc_sh.at[pl.ds(row0 + k * CE, CE), :], rows_v)
        pltpu.sync_copy(rows_v, parts_hbm.at[cid, pl.ds(row0 + k * CE, CE), :])
    pltpu.sync_copy(acc_sh.at[pl.ds(tail0, tail_n), :],
                    rows_v.at[pl.ds(0, tail_n)])
    pltpu.sync_copy(rows_v.at[pl.ds(0, tail_n)],
                    parts_hbm.at[cid, pl.ds(tail0, tail_n), :])


def _edge_scatter(h2d, fsrc3, dst3, w3, zeros_rows):
    mesh = plsc.VectorSubcoreMesh(core_axis_name="c", subcore_axis_name="s")
    return pl.kernel(
        _scatter_body,
        out_type=jax.ShapeDtypeStruct((NC, NP, D), jnp.float32),
        mesh=mesh,
        compiler_params=pltpu.CompilerParams(use_tc_tiling_on_sc=False),
        scratch_types=[
            pltpu.VMEM((NCH, CE), jnp.int32),
            pltpu.VMEM((NCH, CE), jnp.int32),
            pltpu.VMEM((NCH, CE), jnp.float32),
            pltpu.VMEM((CE, D), jnp.float32),
            pltpu.VMEM((CE, D), jnp.float32),
            pltpu.VMEM_SHARED((NP, D), jnp.float32),
            pltpu.SemaphoreType.DMA,
            pltpu.SemaphoreType.DMA,
        ],
    )(h2d, fsrc3, dst3, w3, zeros_rows)


@jax.jit
def kernel(edge_index, edge_type, emb, comp1, bases1, root1, bias1,
           comp2, bases2, root2, bias2):
    src = edge_index[0]
    dst = edge_index[1]

    fsrc3 = (edge_type * N + src).reshape(NW, NCH, CE)
    fdst3 = (edge_type * N + dst).reshape(NW, NCH, CE)
    dst3 = dst.reshape(NW, NCH, CE)

    zeros_cnt = jnp.zeros((CNTP,), jnp.float32)
    zeros_rows = jnp.zeros((CE, D), jnp.float32)

    w3 = _edge_weights(fdst3, zeros_cnt)

    def layer(x, comp, bases, root, bias):
        compf = jnp.zeros((R + 1, R + 1), jnp.float32)
        compf = compf.at[:R, :NB].set(comp).at[R, R].set(1.0)
        basesf = jnp.concatenate([bases, root[None]], axis=0)
        hfull = _matmul_stage(x, compf.reshape(R + 1, 1, R + 1), basesf,
                              bias.reshape(1, D))
        h2d = hfull.reshape((R + 1) * N, D)
        parts = _edge_scatter(h2d, fsrc3, dst3, w3, zeros_rows)
        return _relu_stage(hfull, parts)

    x = layer(emb, comp1, bases1, root1, bias1)
    x = layer(x, comp2, bases2, root2, bias2)
    return x
